# baseline (speedup 1.0000x reference)
import functools

import jax
import jax.numpy as jnp
from jax import lax
from jax.experimental import pallas as pl
from jax.experimental.pallas import tpu as pltpu
from jax.experimental.pallas import tpu_sc as plsc

_K = 1024
_L = 16
_NC = 2
_NS = 16


def _sc_segment_max(imgf, spxf, *, n_rows, hw, n_batch):
    rows_per_w = n_rows // (_NC * _NS)
    nv = hw // _L
    mesh = plsc.VectorSubcoreMesh(core_axis_name="c", subcore_axis_name="s")

    @functools.partial(
        pl.kernel,
        out_type=jax.ShapeDtypeStruct((n_rows, _K), jnp.float32),
        mesh=mesh,
        scratch_types=[
            pltpu.VMEM((hw,), jnp.int32),
            pltpu.VMEM((hw,), jnp.float32),
            pltpu.VMEM((_L * _K,), jnp.float32),
            pltpu.VMEM((_K,), jnp.float32),
        ],
        compiler_params=pltpu.CompilerParams(needs_layout_passes=False),
    )
    def body(img_hbm, spx_hbm, out_hbm, idx_v, img_v, acc_v, out_v):
        cid = lax.axis_index("c")
        sid = lax.axis_index("s")
        wid = sid * _NC + cid
        row0 = wid * rows_per_w
        b = row0 // (n_rows // n_batch)

        pltpu.sync_copy(spx_hbm.at[b], idx_v)
        lane_off = lax.iota(jnp.int32, _L) * _K

        @pl.loop(0, nv)
        def _pre(i):
            s = i * _L
            idx_v[pl.ds(s, _L)] = idx_v[pl.ds(s, _L)] + lane_off

        ninf = jnp.full((_L,), -jnp.inf, dtype=jnp.float32)

        @pl.loop(0, _K)
        def _init(j):
            acc_v[pl.ds(j * _L, _L)] = ninf

        @pl.loop(0, rows_per_w)
        def _row(r):
            row = row0 + r
            pltpu.sync_copy(img_hbm.at[row], img_v)

            @pl.loop(0, nv)
            def _scat(i):
                s = i * _L
                iv = idx_v[pl.ds(s, _L)]
                vals = img_v[pl.ds(s, _L)]
                cur = plsc.load_gather(acc_v, [iv])
                plsc.store_scatter(acc_v, [iv], jnp.maximum(cur, vals))

            @pl.loop(0, _K // _L)
            def _red(q):
                m = acc_v[pl.ds(q * _L, _L)]
                acc_v[pl.ds(q * _L, _L)] = ninf
                for l in range(1, _L):
                    off = l * _K + q * _L
                    m = jnp.maximum(m, acc_v[pl.ds(off, _L)])
                    acc_v[pl.ds(off, _L)] = ninf
                out_v[pl.ds(q * _L, _L)] = m

            pltpu.sync_copy(out_v, out_hbm.at[row])

    return body


def kernel(img, spx):
    B, C, H, W = img.shape
    hw = H * W
    imgf = img.reshape(B * C, hw)
    spxf = spx.reshape(B, hw).astype(jnp.int32)
    out = _sc_segment_max(imgf, spxf, n_rows=B * C, hw=hw, n_batch=B)(
        imgf, spxf)
    return out.reshape(B, C, _K)


# speedup vs baseline: 1.7158x; 1.7158x over previous
import functools

import jax
import jax.numpy as jnp
from jax import lax
from jax.experimental import pallas as pl
from jax.experimental.pallas import tpu as pltpu
from jax.experimental.pallas import tpu_sc as plsc

_K = 1024
_L = 16
_NC = 2
_NS = 16
_NBANK = 2
_NCHUNK = 4


def _sc_segment_max(imgf, spxf, *, n_rows, hw, n_batch):
    rows_per_w = n_rows // (_NC * _NS)
    ch = hw // _NCHUNK
    mesh = plsc.VectorSubcoreMesh(core_axis_name="c", subcore_axis_name="s")

    @functools.partial(
        pl.kernel,
        out_type=jax.ShapeDtypeStruct((n_rows, _K), jnp.float32),
        mesh=mesh,
        scratch_types=[
            pltpu.VMEM((hw,), jnp.int32),
            pltpu.VMEM((ch,), jnp.float32),
            pltpu.VMEM((ch,), jnp.float32),
            pltpu.VMEM((_NBANK * _L * _K,), jnp.float32),
            pltpu.VMEM((_K,), jnp.float32),
            pltpu.SemaphoreType.DMA,
            pltpu.SemaphoreType.DMA,
        ],
        compiler_params=pltpu.CompilerParams(needs_layout_passes=False),
    )
    def body(img_hbm, spx_hbm, out_hbm, idx_v, buf0, buf1, acc_v, out_v,
             sem0, sem1):
        cid = lax.axis_index("c")
        sid = lax.axis_index("s")
        wid = sid * _NC + cid
        row0 = wid * rows_per_w
        b = row0 // (n_rows // n_batch)

        pltpu.sync_copy(spx_hbm.at[b], idx_v)

        lane_off0 = lax.iota(jnp.int32, _L) * _K
        lane_off1 = lane_off0 + _L * _K

        @pl.loop(0, hw // (2 * _L))
        def _pre(i):
            s = i * (2 * _L)
            idx_v[pl.ds(s, _L)] = idx_v[pl.ds(s, _L)] + lane_off0
            idx_v[pl.ds(s + _L, _L)] = idx_v[pl.ds(s + _L, _L)] + lane_off1

        ninf = jnp.full((_L,), -jnp.inf, dtype=jnp.float32)

        @pl.loop(0, _NBANK * _K)
        def _init(j):
            acc_v[pl.ds(j * _L, _L)] = ninf

        bufs = (buf0, buf1)
        sems = (sem0, sem1)
        last_row = n_rows - 1

        pltpu.async_copy(img_hbm.at[row0, pl.ds(0, ch)], buf0, sem0)

        @pl.loop(0, rows_per_w)
        def _row(r):
            row = row0 + r
            nxt = jnp.minimum(row + 1, last_row)
            for c in range(_NCHUNK):
                cur, oth = c % 2, (c + 1) % 2
                if c + 1 < _NCHUNK:
                    pltpu.async_copy(
                        img_hbm.at[row, pl.ds((c + 1) * ch, ch)],
                        bufs[oth], sems[oth])
                else:
                    pltpu.async_copy(
                        img_hbm.at[nxt, pl.ds(0, ch)], bufs[oth], sems[oth])
                pltpu.make_async_copy(
                    img_hbm.at[row, pl.ds(c * ch, ch)],
                    bufs[cur], sems[cur]).wait()

                cbase = c * ch
                buf = bufs[cur]

                @pl.loop(0, ch // (2 * _L), unroll=4)
                def _scat(j):
                    s0 = cbase + j * (2 * _L)
                    l0 = j * (2 * _L)
                    iv0 = idx_v[pl.ds(s0, _L)]
                    iv1 = idx_v[pl.ds(s0 + _L, _L)]
                    v0 = buf[pl.ds(l0, _L)]
                    v1 = buf[pl.ds(l0 + _L, _L)]
                    c0 = plsc.load_gather(acc_v, [iv0])
                    c1 = plsc.load_gather(acc_v, [iv1])
                    plsc.store_scatter(acc_v, [iv0], jnp.maximum(c0, v0))
                    plsc.store_scatter(acc_v, [iv1], jnp.maximum(c1, v1))

            @pl.loop(0, _K // _L)
            def _red(q):
                m = acc_v[pl.ds(q * _L, _L)]
                acc_v[pl.ds(q * _L, _L)] = ninf
                for t in range(1, _NBANK * _L):
                    off = t * _K + q * _L
                    m = jnp.maximum(m, acc_v[pl.ds(off, _L)])
                    acc_v[pl.ds(off, _L)] = ninf
                out_v[pl.ds(q * _L, _L)] = m

            pltpu.sync_copy(out_v, out_hbm.at[row])

        pltpu.make_async_copy(
            img_hbm.at[row0, pl.ds(0, ch)], buf0, sem0).wait()

    return body


def kernel(img, spx):
    B, C, H, W = img.shape
    hw = H * W
    imgf = img.reshape(B * C, hw)
    spxf = spx.reshape(B, hw).astype(jnp.int32)
    out = _sc_segment_max(imgf, spxf, n_rows=B * C, hw=hw, n_batch=B)(
        imgf, spxf)
    return out.reshape(B, C, _K)


# speedup vs baseline: 2.0594x; 1.2002x over previous
import functools

import jax
import jax.numpy as jnp
from jax import lax
from jax.experimental import pallas as pl
from jax.experimental.pallas import tpu as pltpu
from jax.experimental.pallas import tpu_sc as plsc

_K = 1024
_L = 16
_NC = 2
_NS = 16
_NCHUNK = 8
_BANK = _L * _K


def _sc_segment_max(imgf, spxf, *, n_rows, hw, n_batch):
    rows_per_w = n_rows // (_NC * _NS)
    ch = hw // _NCHUNK
    mesh = plsc.VectorSubcoreMesh(core_axis_name="c", subcore_axis_name="s")

    @functools.partial(
        pl.kernel,
        out_type=jax.ShapeDtypeStruct((n_rows, _K), jnp.float32),
        mesh=mesh,
        scratch_types=[
            pltpu.VMEM((hw,), jnp.int16),
            pltpu.VMEM((ch,), jnp.int32),
            pltpu.VMEM((ch,), jnp.float32),
            pltpu.VMEM((ch,), jnp.float32),
            pltpu.VMEM((ch,), jnp.float32),
            pltpu.VMEM((ch,), jnp.float32),
            pltpu.VMEM((4 * _BANK,), jnp.float32),
            pltpu.VMEM((_K,), jnp.float32),
            pltpu.VMEM((_K,), jnp.float32),
            pltpu.SemaphoreType.DMA,
            pltpu.SemaphoreType.DMA,
            pltpu.SemaphoreType.DMA,
            pltpu.SemaphoreType.DMA,
        ],
        compiler_params=pltpu.CompilerParams(needs_layout_passes=False),
    )
    def body(img_hbm, spx_hbm, out_hbm, idx16_v, stage_v,
             bufa0, bufa1, bufb0, bufb1, acc_v, outa_v, outb_v,
             sema0, sema1, semb0, semb1):
        cid = lax.axis_index("c")
        sid = lax.axis_index("s")
        wid = sid * _NC + cid
        row0 = wid * rows_per_w
        b = row0 // (n_rows // n_batch)

        lane_off = lax.iota(jnp.int32, _L) * _K

        for cs in range(_NCHUNK):
            pltpu.sync_copy(spx_hbm.at[b, pl.ds(cs * ch, ch)], stage_v)

            @pl.loop(0, ch // (2 * _L))
            def _pre(j):
                s = j * (2 * _L)
                iv0 = stage_v[pl.ds(s, _L)] + lane_off
                iv1 = stage_v[pl.ds(s + _L, _L)] + lane_off
                ab = plsc.pack(iv0, iv1, format=plsc.PackFormat.INTERLEAVED)
                idx16_v[pl.ds(cs * ch + s, 2 * _L)] = ab

        ninf = jnp.full((_L,), -jnp.inf, dtype=jnp.float32)

        @pl.loop(0, 4 * _K)
        def _init(j):
            acc_v[pl.ds(j * _L, _L)] = ninf

        bufsa = (bufa0, bufa1)
        bufsb = (bufb0, bufb1)
        semsa = (sema0, sema1)
        semsb = (semb0, semb1)

        pltpu.async_copy(img_hbm.at[row0, pl.ds(0, ch)], bufa0, sema0)
        pltpu.async_copy(img_hbm.at[row0 + 1, pl.ds(0, ch)], bufb0, semb0)

        @pl.loop(0, rows_per_w // 2)
        def _pair(p):
            ra = row0 + 2 * p
            rb = ra + 1
            na = jnp.minimum(ra + 2, n_rows - 2)
            nb = na + 1
            for c in range(_NCHUNK):
                cur, oth = c % 2, (c + 1) % 2
                if c + 1 < _NCHUNK:
                    pltpu.async_copy(
                        img_hbm.at[ra, pl.ds((c + 1) * ch, ch)],
                        bufsa[oth], semsa[oth])
                    pltpu.async_copy(
                        img_hbm.at[rb, pl.ds((c + 1) * ch, ch)],
                        bufsb[oth], semsb[oth])
                else:
                    pltpu.async_copy(
                        img_hbm.at[na, pl.ds(0, ch)], bufsa[oth], semsa[oth])
                    pltpu.async_copy(
                        img_hbm.at[nb, pl.ds(0, ch)], bufsb[oth], semsb[oth])
                pltpu.make_async_copy(
                    img_hbm.at[ra, pl.ds(c * ch, ch)],
                    bufsa[cur], semsa[cur]).wait()
                pltpu.make_async_copy(
                    img_hbm.at[rb, pl.ds(c * ch, ch)],
                    bufsb[cur], semsb[cur]).wait()

                cbase = c * ch
                bufa = bufsa[cur]
                bufb = bufsb[cur]

                @pl.loop(0, ch // (2 * _L), unroll=4)
                def _scat(j):
                    s = j * (2 * _L)
                    ab = idx16_v[pl.ds(cbase + s, 2 * _L)]
                    iv0, iv1 = plsc.unpack(
                        ab, format=plsc.PackFormat.INTERLEAVED)
                    ia1 = iv1 + _BANK
                    ib0 = iv0 + 2 * _BANK
                    ib1 = iv1 + 3 * _BANK
                    va0 = bufa[pl.ds(s, _L)]
                    va1 = bufa[pl.ds(s + _L, _L)]
                    vb0 = bufb[pl.ds(s, _L)]
                    vb1 = bufb[pl.ds(s + _L, _L)]
                    c0 = plsc.load_gather(acc_v, [iv0])
                    c1 = plsc.load_gather(acc_v, [ia1])
                    c2 = plsc.load_gather(acc_v, [ib0])
                    c3 = plsc.load_gather(acc_v, [ib1])
                    plsc.store_scatter(acc_v, [iv0], jnp.maximum(c0, va0))
                    plsc.store_scatter(acc_v, [ia1], jnp.maximum(c1, va1))
                    plsc.store_scatter(acc_v, [ib0], jnp.maximum(c2, vb0))
                    plsc.store_scatter(acc_v, [ib1], jnp.maximum(c3, vb1))

            @pl.loop(0, _K // _L)
            def _red(q):
                base = q * _L
                ma = acc_v[pl.ds(base, _L)]
                acc_v[pl.ds(base, _L)] = ninf
                for t in range(1, 2 * _L):
                    off = t * _K + base
                    ma = jnp.maximum(ma, acc_v[pl.ds(off, _L)])
                    acc_v[pl.ds(off, _L)] = ninf
                outa_v[pl.ds(base, _L)] = ma
                mb = acc_v[pl.ds(2 * _BANK + base, _L)]
                acc_v[pl.ds(2 * _BANK + base, _L)] = ninf
                for t in range(1, 2 * _L):
                    off = 2 * _BANK + t * _K + base
                    mb = jnp.maximum(mb, acc_v[pl.ds(off, _L)])
                    acc_v[pl.ds(off, _L)] = ninf
                outb_v[pl.ds(base, _L)] = mb

            pltpu.sync_copy(outa_v, out_hbm.at[ra])
            pltpu.sync_copy(outb_v, out_hbm.at[rb])

        pltpu.make_async_copy(
            img_hbm.at[row0, pl.ds(0, ch)], bufa0, sema0).wait()
        pltpu.make_async_copy(
            img_hbm.at[row0 + 1, pl.ds(0, ch)], bufb0, semb0).wait()

    return body


def kernel(img, spx):
    B, C, H, W = img.shape
    hw = H * W
    imgf = img.reshape(B * C, hw)
    spxf = spx.reshape(B, hw).astype(jnp.int32)
    out = _sc_segment_max(imgf, spxf, n_rows=B * C, hw=hw, n_batch=B)(
        imgf, spxf)
    return out.reshape(B, C, _K)


# speedup vs baseline: 2.0747x; 1.0074x over previous
import functools

import jax
import jax.numpy as jnp
from jax import lax
from jax.experimental import pallas as pl
from jax.experimental.pallas import tpu as pltpu
from jax.experimental.pallas import tpu_sc as plsc

_K = 1024
_L = 16
_NC = 2
_NS = 16
_NCHUNK = 8
_BANK = _L * _K


def _sc_segment_max(imgf, spxf, *, n_rows, hw, n_batch):
    rows_per_w = n_rows // (_NC * _NS)
    ch = hw // _NCHUNK
    mesh = plsc.VectorSubcoreMesh(core_axis_name="c", subcore_axis_name="s")

    @functools.partial(
        pl.kernel,
        out_type=jax.ShapeDtypeStruct((n_rows, _K), jnp.float32),
        mesh=mesh,
        scratch_types=[
            pltpu.VMEM((hw // 2,), jnp.int32),
            pltpu.VMEM((ch,), jnp.int32),
            pltpu.VMEM((ch,), jnp.float32),
            pltpu.VMEM((ch,), jnp.float32),
            pltpu.VMEM((ch,), jnp.float32),
            pltpu.VMEM((ch,), jnp.float32),
            pltpu.VMEM((4 * _BANK,), jnp.float32),
            pltpu.VMEM((_K,), jnp.float32),
            pltpu.VMEM((_K,), jnp.float32),
            pltpu.SemaphoreType.DMA,
            pltpu.SemaphoreType.DMA,
            pltpu.SemaphoreType.DMA,
            pltpu.SemaphoreType.DMA,
        ],
        compiler_params=pltpu.CompilerParams(needs_layout_passes=False),
    )
    def body(img_hbm, spx_hbm, out_hbm, idxp_v, stage_v,
             bufa0, bufa1, bufb0, bufb1, acc_v, outa_v, outb_v,
             sema0, sema1, semb0, semb1):
        cid = lax.axis_index("c")
        sid = lax.axis_index("s")
        wid = sid * _NC + cid
        row0 = wid * rows_per_w
        b = row0 // (n_rows // n_batch)

        lane_off = lax.iota(jnp.int32, _L) * _K
        lane_off_hi = lane_off + _BANK

        for cs in range(_NCHUNK):
            pltpu.sync_copy(spx_hbm.at[b, pl.ds(cs * ch, ch)], stage_v)

            @pl.loop(0, ch // (2 * _L))
            def _pre(j):
                s = j * (2 * _L)
                lo = stage_v[pl.ds(s, _L)] + lane_off
                hi = stage_v[pl.ds(s + _L, _L)] + lane_off_hi
                idxp_v[pl.ds((cs * ch + s) // 2, _L)] = (
                    lo | (hi << 16))

        ninf = jnp.full((_L,), -jnp.inf, dtype=jnp.float32)

        @pl.loop(0, 4 * _K)
        def _init(j):
            acc_v[pl.ds(j * _L, _L)] = ninf

        bufsa = (bufa0, bufa1)
        bufsb = (bufb0, bufb1)
        semsa = (sema0, sema1)
        semsb = (semb0, semb1)

        pltpu.async_copy(img_hbm.at[row0, pl.ds(0, ch)], bufa0, sema0)
        pltpu.async_copy(img_hbm.at[row0 + 1, pl.ds(0, ch)], bufb0, semb0)

        @pl.loop(0, rows_per_w // 2)
        def _pair(p):
            ra = row0 + 2 * p
            rb = ra + 1
            na = jnp.minimum(ra + 2, n_rows - 2)
            nb = na + 1
            for c in range(_NCHUNK):
                cur, oth = c % 2, (c + 1) % 2
                if c + 1 < _NCHUNK:
                    pltpu.async_copy(
                        img_hbm.at[ra, pl.ds((c + 1) * ch, ch)],
                        bufsa[oth], semsa[oth])
                    pltpu.async_copy(
                        img_hbm.at[rb, pl.ds((c + 1) * ch, ch)],
                        bufsb[oth], semsb[oth])
                else:
                    pltpu.async_copy(
                        img_hbm.at[na, pl.ds(0, ch)], bufsa[oth], semsa[oth])
                    pltpu.async_copy(
                        img_hbm.at[nb, pl.ds(0, ch)], bufsb[oth], semsb[oth])
                pltpu.make_async_copy(
                    img_hbm.at[ra, pl.ds(c * ch, ch)],
                    bufsa[cur], semsa[cur]).wait()
                pltpu.make_async_copy(
                    img_hbm.at[rb, pl.ds(c * ch, ch)],
                    bufsb[cur], semsb[cur]).wait()

                cb2 = (c * ch) // 2
                bufa = bufsa[cur]
                bufb = bufsb[cur]

                @pl.loop(0, ch // (2 * _L), unroll=4)
                def _scat(j):
                    s = j * (2 * _L)
                    cw = idxp_v[pl.ds(cb2 + j * _L, _L)]
                    iv0 = cw & 0xFFFF
                    ia1 = lax.shift_right_logical(cw, 16)
                    ib0 = iv0 + 2 * _BANK
                    ib1 = ia1 + 2 * _BANK
                    va0 = bufa[pl.ds(s, _L)]
                    va1 = bufa[pl.ds(s + _L, _L)]
                    vb0 = bufb[pl.ds(s, _L)]
                    vb1 = bufb[pl.ds(s + _L, _L)]
                    c0 = plsc.load_gather(acc_v, [iv0])
                    c1 = plsc.load_gather(acc_v, [ia1])
                    c2 = plsc.load_gather(acc_v, [ib0])
                    c3 = plsc.load_gather(acc_v, [ib1])
                    plsc.store_scatter(acc_v, [iv0], jnp.maximum(c0, va0))
                    plsc.store_scatter(acc_v, [ia1], jnp.maximum(c1, va1))
                    plsc.store_scatter(acc_v, [ib0], jnp.maximum(c2, vb0))
                    plsc.store_scatter(acc_v, [ib1], jnp.maximum(c3, vb1))

            @pl.loop(0, _K // _L)
            def _red(q):
                base = q * _L
                ma = acc_v[pl.ds(base, _L)]
                acc_v[pl.ds(base, _L)] = ninf
                for t in range(1, 2 * _L):
                    off = t * _K + base
                    ma = jnp.maximum(ma, acc_v[pl.ds(off, _L)])
                    acc_v[pl.ds(off, _L)] = ninf
                outa_v[pl.ds(base, _L)] = ma
                mb = acc_v[pl.ds(2 * _BANK + base, _L)]
                acc_v[pl.ds(2 * _BANK + base, _L)] = ninf
                for t in range(1, 2 * _L):
                    off = 2 * _BANK + t * _K + base
                    mb = jnp.maximum(mb, acc_v[pl.ds(off, _L)])
                    acc_v[pl.ds(off, _L)] = ninf
                outb_v[pl.ds(base, _L)] = mb

            pltpu.sync_copy(outa_v, out_hbm.at[ra])
            pltpu.sync_copy(outb_v, out_hbm.at[rb])

        pltpu.make_async_copy(
            img_hbm.at[row0, pl.ds(0, ch)], bufa0, sema0).wait()
        pltpu.make_async_copy(
            img_hbm.at[row0 + 1, pl.ds(0, ch)], bufb0, semb0).wait()

    return body


def kernel(img, spx):
    B, C, H, W = img.shape
    hw = H * W
    imgf = img.reshape(B * C, hw)
    spxf = spx.reshape(B, hw).astype(jnp.int32)
    out = _sc_segment_max(imgf, spxf, n_rows=B * C, hw=hw, n_batch=B)(
        imgf, spxf)
    return out.reshape(B, C, _K)


# speedup vs baseline: 2.3809x; 1.1476x over previous
import functools

import jax
import jax.numpy as jnp
from jax import lax
from jax.experimental import pallas as pl
from jax.experimental.pallas import tpu as pltpu
from jax.experimental.pallas import tpu_sc as plsc

_K = 1024
_L = 16
_NC = 2
_NS = 16
_NCHUNK = 14
_GROUP = 3
_BANK = _L * _K


def _sc_segment_max(imgf, spxf, *, n_rows, hw, n_batch):
    rows_per_w = n_rows // (_NC * _NS)
    ch = hw // _NCHUNK
    mesh = plsc.VectorSubcoreMesh(core_axis_name="c", subcore_axis_name="s")

    @functools.partial(
        pl.kernel,
        out_type=jax.ShapeDtypeStruct((n_rows, _K), jnp.float32),
        mesh=mesh,
        scratch_types=[
            pltpu.VMEM((ch,), jnp.int32),
            pltpu.VMEM((ch,), jnp.int32),
            pltpu.VMEM((ch,), jnp.float32),
            pltpu.VMEM((ch,), jnp.float32),
            pltpu.VMEM((ch,), jnp.float32),
            pltpu.VMEM((ch,), jnp.float32),
            pltpu.VMEM((ch,), jnp.float32),
            pltpu.VMEM((ch,), jnp.float32),
            pltpu.VMEM((2 * _GROUP * _BANK,), jnp.float32),
            pltpu.VMEM((_K,), jnp.float32),
            pltpu.VMEM((_K,), jnp.float32),
            pltpu.VMEM((_K,), jnp.float32),
            pltpu.SemaphoreType.DMA,
            pltpu.SemaphoreType.DMA,
            pltpu.SemaphoreType.DMA,
            pltpu.SemaphoreType.DMA,
            pltpu.SemaphoreType.DMA,
            pltpu.SemaphoreType.DMA,
            pltpu.SemaphoreType.DMA,
            pltpu.SemaphoreType.DMA,
        ],
        compiler_params=pltpu.CompilerParams(needs_layout_passes=False),
    )
    def body(img_hbm, spx_hbm, out_hbm, idx0, idx1,
             bufa0, bufa1, bufb0, bufb1, bufc0, bufc1,
             acc_v, outa_v, outb_v, outc_v,
             isem0, isem1, sa0, sa1, sb0, sb1, sc0, sc1):
        cid = lax.axis_index("c")
        sid = lax.axis_index("s")
        wid = sid * _NC + cid
        row0 = wid * rows_per_w
        b = row0 // (n_rows // n_batch)

        lane0 = lax.iota(jnp.int32, _L) * _K
        lane1 = lane0 + _BANK

        ninf = jnp.full((_L,), -jnp.inf, dtype=jnp.float32)

        @pl.loop(0, 2 * _GROUP * _K)
        def _init(j):
            acc_v[pl.ds(j * _L, _L)] = ninf

        idxs = (idx0, idx1)
        isems = (isem0, isem1)
        bufs = ((bufa0, bufa1), (bufb0, bufb1), (bufc0, bufc1))
        sems = ((sa0, sa1), (sb0, sb1), (sc0, sc1))
        outs = (outa_v, outb_v, outc_v)

        def issue(r0, c, par):
            pltpu.async_copy(
                spx_hbm.at[b, pl.ds(c * ch, ch)], idxs[par], isems[par])
            for k in range(_GROUP):
                pltpu.async_copy(
                    img_hbm.at[r0 + k, pl.ds(c * ch, ch)],
                    bufs[k][par], sems[k][par])

        def wait(r0, c, par):
            pltpu.make_async_copy(
                spx_hbm.at[b, pl.ds(c * ch, ch)],
                idxs[par], isems[par]).wait()
            for k in range(_GROUP):
                pltpu.make_async_copy(
                    img_hbm.at[r0 + k, pl.ds(c * ch, ch)],
                    bufs[k][par], sems[k][par]).wait()

        issue(row0, 0, 0)

        @pl.loop(0, rows_per_w // _GROUP)
        def _grp(g):
            r0 = row0 + _GROUP * g
            nr0 = jnp.minimum(r0 + _GROUP, n_rows - _GROUP)
            for c in range(_NCHUNK):
                cur, oth = c % 2, (c + 1) % 2
                if c + 1 < _NCHUNK:
                    issue(r0, c + 1, oth)
                else:
                    issue(nr0, 0, oth)
                wait(r0, c, cur)

                ib = idxs[cur]
                ba, bb, bc = bufs[0][cur], bufs[1][cur], bufs[2][cur]

                @pl.loop(0, ch // (2 * _L), unroll=2)
                def _scat(j):
                    s = j * (2 * _L)
                    i0 = ib[pl.ds(s, _L)]
                    i1 = ib[pl.ds(s + _L, _L)]
                    a0 = i0 + lane0
                    a1 = i1 + lane1
                    b0 = a0 + 2 * _BANK
                    b1 = a1 + 2 * _BANK
                    c0 = a0 + 4 * _BANK
                    c1 = a1 + 4 * _BANK
                    va0 = ba[pl.ds(s, _L)]
                    va1 = ba[pl.ds(s + _L, _L)]
                    vb0 = bb[pl.ds(s, _L)]
                    vb1 = bb[pl.ds(s + _L, _L)]
                    vc0 = bc[pl.ds(s, _L)]
                    vc1 = bc[pl.ds(s + _L, _L)]
                    g0 = plsc.load_gather(acc_v, [a0])
                    g1 = plsc.load_gather(acc_v, [a1])
                    g2 = plsc.load_gather(acc_v, [b0])
                    g3 = plsc.load_gather(acc_v, [b1])
                    g4 = plsc.load_gather(acc_v, [c0])
                    g5 = plsc.load_gather(acc_v, [c1])
                    plsc.store_scatter(acc_v, [a0], jnp.maximum(g0, va0))
                    plsc.store_scatter(acc_v, [a1], jnp.maximum(g1, va1))
                    plsc.store_scatter(acc_v, [b0], jnp.maximum(g2, vb0))
                    plsc.store_scatter(acc_v, [b1], jnp.maximum(g3, vb1))
                    plsc.store_scatter(acc_v, [c0], jnp.maximum(g4, vc0))
                    plsc.store_scatter(acc_v, [c1], jnp.maximum(g5, vc1))

            @pl.loop(0, _K // _L)
            def _red(q):
                base = q * _L
                for k in range(_GROUP):
                    boff = 2 * k * _BANK
                    m = acc_v[pl.ds(boff + base, _L)]
                    acc_v[pl.ds(boff + base, _L)] = ninf
                    for t in range(1, 2 * _L):
                        off = boff + t * _K + base
                        m = jnp.maximum(m, acc_v[pl.ds(off, _L)])
                        acc_v[pl.ds(off, _L)] = ninf
                    outs[k][pl.ds(base, _L)] = m

            for k in range(_GROUP):
                pltpu.sync_copy(outs[k], out_hbm.at[r0 + k])

        wait(row0, 0, 0)

    return body


def kernel(img, spx):
    B, C, H, W = img.shape
    hw = H * W
    imgf = img.reshape(B * C, hw)
    spxf = spx.reshape(B, hw).astype(jnp.int32)
    out = _sc_segment_max(imgf, spxf, n_rows=B * C, hw=hw, n_batch=B)(
        imgf, spxf)
    return out.reshape(B, C, _K)


# speedup vs baseline: 2.5318x; 1.0634x over previous
import functools

import jax
import jax.numpy as jnp
from jax import lax
from jax.experimental import pallas as pl
from jax.experimental.pallas import tpu as pltpu
from jax.experimental.pallas import tpu_sc as plsc

_K = 1024
_L = 16
_NC = 2
_NS = 16
_NCHUNK = 14
_GROUP = 3
_BANK = _L * _K


def _sc_segment_max(imgf, spxf, *, n_rows, hw, n_batch):
    rows_per_w = n_rows // (_NC * _NS)
    ch = hw // _NCHUNK
    mesh = plsc.VectorSubcoreMesh(core_axis_name="c", subcore_axis_name="s")

    @functools.partial(
        pl.kernel,
        out_type=jax.ShapeDtypeStruct((n_rows, _K), jnp.float32),
        mesh=mesh,
        scratch_types=[
            pltpu.VMEM((ch,), jnp.int32),
            pltpu.VMEM((ch,), jnp.int32),
            pltpu.VMEM((ch,), jnp.float32),
            pltpu.VMEM((ch,), jnp.float32),
            pltpu.VMEM((ch,), jnp.float32),
            pltpu.VMEM((ch,), jnp.float32),
            pltpu.VMEM((ch,), jnp.float32),
            pltpu.VMEM((ch,), jnp.float32),
            pltpu.VMEM((2 * _GROUP * _BANK,), jnp.float32),
            pltpu.VMEM((_K,), jnp.float32),
            pltpu.VMEM((_K,), jnp.float32),
            pltpu.VMEM((_K,), jnp.float32),
            pltpu.SemaphoreType.DMA,
            pltpu.SemaphoreType.DMA,
            pltpu.SemaphoreType.DMA,
            pltpu.SemaphoreType.DMA,
            pltpu.SemaphoreType.DMA,
            pltpu.SemaphoreType.DMA,
            pltpu.SemaphoreType.DMA,
            pltpu.SemaphoreType.DMA,
        ],
        compiler_params=pltpu.CompilerParams(needs_layout_passes=False),
    )
    def body(img_hbm, spx_hbm, out_hbm, idx0, idx1,
             bufa0, bufa1, bufb0, bufb1, bufc0, bufc1,
             acc_v, outa_v, outb_v, outc_v,
             isem0, isem1, sa0, sa1, sb0, sb1, sc0, sc1):
        cid = lax.axis_index("c")
        sid = lax.axis_index("s")
        wid = sid * _NC + cid
        row0 = wid * rows_per_w
        b = row0 // (n_rows // n_batch)

        lane0 = lax.iota(jnp.int32, _L) * _K
        lane1 = lane0 + _BANK

        ninf = jnp.full((_L,), -jnp.inf, dtype=jnp.float32)

        @pl.loop(0, 2 * _GROUP * _K)
        def _init(j):
            acc_v[pl.ds(j * _L, _L)] = ninf

        idxs = (idx0, idx1)
        isems = (isem0, isem1)
        bufs = ((bufa0, bufa1), (bufb0, bufb1), (bufc0, bufc1))
        sems = ((sa0, sa1), (sb0, sb1), (sc0, sc1))
        outs = (outa_v, outb_v, outc_v)

        def issue(r0, c, par):
            pltpu.async_copy(
                spx_hbm.at[b, pl.ds(c * ch, ch)], idxs[par], isems[par])
            for k in range(_GROUP):
                pltpu.async_copy(
                    img_hbm.at[r0 + k, pl.ds(c * ch, ch)],
                    bufs[k][par], sems[k][par])

        def wait(r0, c, par):
            pltpu.make_async_copy(
                spx_hbm.at[b, pl.ds(c * ch, ch)],
                idxs[par], isems[par]).wait()
            for k in range(_GROUP):
                pltpu.make_async_copy(
                    img_hbm.at[r0 + k, pl.ds(c * ch, ch)],
                    bufs[k][par], sems[k][par]).wait()

        issue(row0, 0, 0)

        @pl.loop(0, rows_per_w // _GROUP)
        def _grp(g):
            r0 = row0 + _GROUP * g
            nr0 = jnp.minimum(r0 + _GROUP, n_rows - _GROUP)
            for c in range(_NCHUNK):
                cur, oth = c % 2, (c + 1) % 2
                if c + 1 < _NCHUNK:
                    issue(r0, c + 1, oth)
                else:
                    issue(nr0, 0, oth)
                wait(r0, c, cur)

                ib = idxs[cur]
                ba, bb, bc = bufs[0][cur], bufs[1][cur], bufs[2][cur]

                @pl.loop(0, ch // (4 * _L), unroll=2)
                def _scat(j):
                    s = j * (4 * _L)
                    i0 = ib[pl.ds(s, _L)] + lane0
                    i1 = ib[pl.ds(s + _L, _L)] + lane1
                    i2 = ib[pl.ds(s + 2 * _L, _L)] + lane0
                    i3 = ib[pl.ds(s + 3 * _L, _L)] + lane1
                    m02 = i0 == i2
                    m13 = i1 == i3
                    addrs = []
                    vals = []
                    gaths = []
                    for k, buf in enumerate((ba, bb, bc)):
                        o = 2 * k * _BANK
                        p = (i0 + o, i1 + o, i2 + o, i3 + o)
                        v = tuple(buf[pl.ds(s + t * _L, _L)]
                                  for t in range(4))
                        addrs.append(p)
                        vals.append(v)
                        gaths.append(tuple(
                            plsc.load_gather(acc_v, [p[t]])
                            for t in range(4)))
                    for k in range(_GROUP):
                        p = addrs[k]
                        v = vals[k]
                        g = gaths[k]
                        v2 = jnp.where(m02, jnp.maximum(v[0], v[2]), v[2])
                        v3 = jnp.where(m13, jnp.maximum(v[1], v[3]), v[3])
                        plsc.store_scatter(acc_v, [p[0]],
                                           jnp.maximum(g[0], v[0]))
                        plsc.store_scatter(acc_v, [p[1]],
                                           jnp.maximum(g[1], v[1]))
                        plsc.store_scatter(acc_v, [p[2]],
                                           jnp.maximum(g[2], v2))
                        plsc.store_scatter(acc_v, [p[3]],
                                           jnp.maximum(g[3], v3))

            @pl.loop(0, _K // _L)
            def _red(q):
                base = q * _L
                for k in range(_GROUP):
                    boff = 2 * k * _BANK
                    m = acc_v[pl.ds(boff + base, _L)]
                    acc_v[pl.ds(boff + base, _L)] = ninf
                    for t in range(1, 2 * _L):
                        off = boff + t * _K + base
                        m = jnp.maximum(m, acc_v[pl.ds(off, _L)])
                        acc_v[pl.ds(off, _L)] = ninf
                    outs[k][pl.ds(base, _L)] = m

            for k in range(_GROUP):
                pltpu.sync_copy(outs[k], out_hbm.at[r0 + k])

        wait(row0, 0, 0)

    return body


def kernel(img, spx):
    B, C, H, W = img.shape
    hw = H * W
    imgf = img.reshape(B * C, hw)
    spxf = spx.reshape(B, hw).astype(jnp.int32)
    out = _sc_segment_max(imgf, spxf, n_rows=B * C, hw=hw, n_batch=B)(
        imgf, spxf)
    return out.reshape(B, C, _K)


# speedup vs baseline: 2.5872x; 1.0219x over previous
import functools

import jax
import jax.numpy as jnp
from jax import lax
from jax.experimental import pallas as pl
from jax.experimental.pallas import tpu as pltpu
from jax.experimental.pallas import tpu_sc as plsc

_K = 1024
_L = 16
_NC = 2
_NS = 16
_NCHUNK = 14
_GROUP = 3
_BANK = _L * _K


def _sc_segment_max(imgf, spxf, *, n_rows, hw, n_batch):
    rows_per_w = n_rows // (_NC * _NS)
    ch = hw // _NCHUNK
    mesh = plsc.VectorSubcoreMesh(core_axis_name="c", subcore_axis_name="s")

    @functools.partial(
        pl.kernel,
        out_type=jax.ShapeDtypeStruct((n_rows, _K), jnp.float32),
        mesh=mesh,
        scratch_types=[
            pltpu.VMEM((ch,), jnp.int32),
            pltpu.VMEM((ch,), jnp.int32),
            pltpu.VMEM((ch,), jnp.float32),
            pltpu.VMEM((ch,), jnp.float32),
            pltpu.VMEM((ch,), jnp.float32),
            pltpu.VMEM((ch,), jnp.float32),
            pltpu.VMEM((ch,), jnp.float32),
            pltpu.VMEM((ch,), jnp.float32),
            pltpu.VMEM((_GROUP * _BANK,), jnp.float32),
            pltpu.VMEM((_K,), jnp.float32),
            pltpu.VMEM((_K,), jnp.float32),
            pltpu.VMEM((_K,), jnp.float32),
            pltpu.SemaphoreType.DMA,
            pltpu.SemaphoreType.DMA,
            pltpu.SemaphoreType.DMA,
            pltpu.SemaphoreType.DMA,
            pltpu.SemaphoreType.DMA,
            pltpu.SemaphoreType.DMA,
            pltpu.SemaphoreType.DMA,
            pltpu.SemaphoreType.DMA,
        ],
        compiler_params=pltpu.CompilerParams(needs_layout_passes=False),
    )
    def body(img_hbm, spx_hbm, out_hbm, idx0, idx1,
             bufa0, bufa1, bufb0, bufb1, bufc0, bufc1,
             acc_v, outa_v, outb_v, outc_v,
             isem0, isem1, sa0, sa1, sb0, sb1, sc0, sc1):
        cid = lax.axis_index("c")
        sid = lax.axis_index("s")
        wid = sid * _NC + cid
        row0 = wid * rows_per_w
        b = row0 // (n_rows // n_batch)

        lane = lax.iota(jnp.int32, _L)
        lane_b = lane + _BANK
        lane16 = lane * _L

        ninf = jnp.full((_L,), -jnp.inf, dtype=jnp.float32)

        @pl.loop(0, _GROUP * _K)
        def _init(j):
            acc_v[pl.ds(j * _L, _L)] = ninf

        idxs = (idx0, idx1)
        isems = (isem0, isem1)
        bufs = ((bufa0, bufa1), (bufb0, bufb1), (bufc0, bufc1))
        sems = ((sa0, sa1), (sb0, sb1), (sc0, sc1))
        outs = (outa_v, outb_v, outc_v)

        def issue(r0, c, par):
            pltpu.async_copy(
                spx_hbm.at[b, pl.ds(c * ch, ch)], idxs[par], isems[par])
            for k in range(_GROUP):
                pltpu.async_copy(
                    img_hbm.at[r0 + k, pl.ds(c * ch, ch)],
                    bufs[k][par], sems[k][par])

        def wait(r0, c, par):
            pltpu.make_async_copy(
                spx_hbm.at[b, pl.ds(c * ch, ch)],
                idxs[par], isems[par]).wait()
            for k in range(_GROUP):
                pltpu.make_async_copy(
                    img_hbm.at[r0 + k, pl.ds(c * ch, ch)],
                    bufs[k][par], sems[k][par]).wait()

        issue(row0, 0, 0)

        @pl.loop(0, rows_per_w // _GROUP)
        def _grp(g):
            r0 = row0 + _GROUP * g
            nr0 = jnp.minimum(r0 + _GROUP, n_rows - _GROUP)
            for c in range(_NCHUNK):
                cur, oth = c % 2, (c + 1) % 2
                if c + 1 < _NCHUNK:
                    issue(r0, c + 1, oth)
                else:
                    issue(nr0, 0, oth)
                wait(r0, c, cur)

                ib = idxs[cur]
                ba, bb, bc = bufs[0][cur], bufs[1][cur], bufs[2][cur]

                @pl.loop(0, ch // (4 * _L), unroll=2)
                def _scat(j):
                    s = j * (4 * _L)
                    i0 = ib[pl.ds(s, _L)] * _L + lane
                    i1 = ib[pl.ds(s + _L, _L)] * _L + lane
                    i2 = ib[pl.ds(s + 2 * _L, _L)] * _L + lane
                    i3 = ib[pl.ds(s + 3 * _L, _L)] * _L + lane
                    m10 = i1 == i0
                    m20 = i2 == i0
                    m21 = i2 == i1
                    m30 = i3 == i0
                    m31 = i3 == i1
                    m32 = i3 == i2
                    addrs = []
                    vals = []
                    gaths = []
                    for k, buf in enumerate((ba, bb, bc)):
                        o = k * _BANK
                        p = (i0 + o, i1 + o, i2 + o, i3 + o)
                        v = tuple(buf[pl.ds(s + t * _L, _L)]
                                  for t in range(4))
                        addrs.append(p)
                        vals.append(v)
                        gaths.append(tuple(
                            plsc.load_gather(acc_v, [p[t]])
                            for t in range(4)))
                    for k in range(_GROUP):
                        p = addrs[k]
                        v = vals[k]
                        g = gaths[k]
                        v1 = jnp.where(m10, jnp.maximum(v[1], v[0]), v[1])
                        v2 = jnp.where(m20, jnp.maximum(v[2], v[0]), v[2])
                        v2 = jnp.where(m21, jnp.maximum(v2, v[1]), v2)
                        v3 = jnp.where(m30, jnp.maximum(v[3], v[0]), v[3])
                        v3 = jnp.where(m31, jnp.maximum(v3, v[1]), v3)
                        v3 = jnp.where(m32, jnp.maximum(v3, v[2]), v3)
                        plsc.store_scatter(acc_v, [p[0]],
                                           jnp.maximum(g[0], v[0]))
                        plsc.store_scatter(acc_v, [p[1]],
                                           jnp.maximum(g[1], v1))
                        plsc.store_scatter(acc_v, [p[2]],
                                           jnp.maximum(g[2], v2))
                        plsc.store_scatter(acc_v, [p[3]],
                                           jnp.maximum(g[3], v3))

            @pl.loop(0, _K // _L)
            def _red(q):
                base = lane16 + q * (_L * _L)
                for k in range(_GROUP):
                    boff = k * _BANK
                    rot = lane
                    m = ninf
                    for t in range(_L):
                        g = plsc.load_gather(acc_v, [base + rot + boff])
                        m = jnp.maximum(m, g)
                        if t + 1 < _L:
                            rot = (rot + 1) & 15
                    outs[k][pl.ds(q * _L, _L)] = m

            @pl.loop(0, _GROUP * _K, unroll=4)
            def _reinit(j):
                acc_v[pl.ds(j * _L, _L)] = ninf

            for k in range(_GROUP):
                pltpu.sync_copy(outs[k], out_hbm.at[r0 + k])

        wait(row0, 0, 0)

    return body


def kernel(img, spx):
    B, C, H, W = img.shape
    hw = H * W
    imgf = img.reshape(B * C, hw)
    spxf = spx.reshape(B, hw).astype(jnp.int32)
    out = _sc_segment_max(imgf, spxf, n_rows=B * C, hw=hw, n_batch=B)(
        imgf, spxf)
    return out.reshape(B, C, _K)
